# initial kernel scaffold (unmeasured)
import jax
import jax.numpy as jnp
from jax import lax
from jax.experimental import pallas as pl
from jax.experimental.pallas import tpu as pltpu

N_DEV = 4


def kernel(x, dy):
    m, d = x.shape
    _, f = dy.shape
    chunk = d // N_DEV

    def body(x_ref, dy_ref, out_ref, acc_ref, comm_ref, send_sems, recv_sems):
        my = lax.axis_index("i")
        left = lax.rem(my + N_DEV - 1, N_DEV)
        right = lax.rem(my + 1, N_DEV)

        barrier_sem = pltpu.get_barrier_semaphore()
        for nbr in [left, right]:
            pl.semaphore_signal(
                barrier_sem, inc=1,
                device_id=(nbr,), device_id_type=pl.DeviceIdType.MESH,
            )
        pl.semaphore_wait(barrier_sem, 2)

        xb = x_ref[...].astype(jnp.bfloat16)
        dyb = dy_ref[...].astype(jnp.bfloat16)
        acc_ref[...] = lax.dot_general(
            xb, dyb,
            dimension_numbers=(((0,), (0,)), ((), ())),
            preferred_element_type=jnp.float32,
        )

        for h in range(N_DEV - 1):
            sc = lax.rem(my + 2 * N_DEV - 1 - h, N_DEV)
            rdma = pltpu.make_async_remote_copy(
                src_ref=acc_ref.at[pl.ds(sc * chunk, chunk), :],
                dst_ref=comm_ref.at[h],
                send_sem=send_sems.at[h],
                recv_sem=recv_sems.at[h],
                device_id=(right,),
                device_id_type=pl.DeviceIdType.MESH,
            )
            rdma.start()
            rdma.wait()

            rc = lax.rem(my + 2 * N_DEV - 2 - h, N_DEV)
            acc_ref[pl.ds(rc * chunk, chunk), :] = (
                acc_ref[pl.ds(rc * chunk, chunk), :] + comm_ref[h]
            )

        out_ref[...] = acc_ref[pl.ds(my * chunk, chunk), :]

    return pl.pallas_call(
        body,
        out_shape=jax.ShapeDtypeStruct((chunk, f), jnp.float32),
        in_specs=[
            pl.BlockSpec(memory_space=pltpu.VMEM),
            pl.BlockSpec(memory_space=pltpu.VMEM),
        ],
        out_specs=pl.BlockSpec(memory_space=pltpu.VMEM),
        scratch_shapes=[
            pltpu.VMEM((d, f), jnp.float32),
            pltpu.VMEM((N_DEV - 1, chunk, f), jnp.float32),
            pltpu.SemaphoreType.DMA((N_DEV - 1,)),
            pltpu.SemaphoreType.DMA((N_DEV - 1,)),
        ],
        compiler_params=pltpu.CompilerParams(collective_id=0),
    )(x, dy)


# baseline (device time: 167600 ns/iter reference)
import jax
import jax.numpy as jnp
from jax import lax
from jax.experimental import pallas as pl
from jax.experimental.pallas import tpu as pltpu

N_DEV = 4


def kernel(x, dy):
    m, d = x.shape
    _, f = dy.shape
    chunk = d // N_DEV

    def body(x_ref, dy_ref, out_ref, acc_ref, comm_ref, send_sems, recv_sems):
        my = lax.axis_index("i")
        left = lax.rem(my + N_DEV - 1, N_DEV)
        right = lax.rem(my + 1, N_DEV)

        barrier_sem = pltpu.get_barrier_semaphore()
        for nbr in [left, right]:
            pl.semaphore_signal(
                barrier_sem, inc=1,
                device_id=(nbr,), device_id_type=pl.DeviceIdType.MESH,
            )
        pl.semaphore_wait(barrier_sem, 2)

        xb = x_ref[...].astype(jnp.bfloat16)
        dyb = dy_ref[...].astype(jnp.bfloat16)
        acc_ref[...] = lax.dot_general(
            xb, dyb,
            dimension_numbers=(((0,), (0,)), ((), ())),
            preferred_element_type=jnp.float32,
        )

        for h in range(N_DEV - 1):
            sc = lax.rem(my + 2 * N_DEV - 1 - h, N_DEV)
            rdma = pltpu.make_async_remote_copy(
                src_ref=acc_ref.at[pl.ds(sc * chunk, chunk), :],
                dst_ref=comm_ref.at[h],
                send_sem=send_sems.at[h],
                recv_sem=recv_sems.at[h],
                device_id=(right,),
                device_id_type=pl.DeviceIdType.MESH,
            )
            rdma.start()
            rdma.wait()

            rc = lax.rem(my + 2 * N_DEV - 2 - h, N_DEV)
            acc_ref[pl.ds(rc * chunk, chunk), :] = (
                acc_ref[pl.ds(rc * chunk, chunk), :] + comm_ref[h]
            )

        out_ref[...] = acc_ref[pl.ds(my * chunk, chunk), :]

    return pl.pallas_call(
        body,
        out_shape=jax.ShapeDtypeStruct((chunk, f), jnp.float32),
        in_specs=[
            pl.BlockSpec(memory_space=pltpu.VMEM),
            pl.BlockSpec(memory_space=pltpu.VMEM),
        ],
        out_specs=pl.BlockSpec(memory_space=pltpu.VMEM),
        scratch_shapes=[
            pltpu.VMEM((d, f), jnp.float32),
            pltpu.VMEM((N_DEV - 1, chunk, f), jnp.float32),
            pltpu.SemaphoreType.DMA((N_DEV - 1,)),
            pltpu.SemaphoreType.DMA((N_DEV - 1,)),
        ],
        compiler_params=pltpu.CompilerParams(
            collective_id=0,
            vmem_limit_bytes=100 * 1024 * 1024,
        ),
    )(x, dy)


# device time: 66615 ns/iter; 2.5159x vs baseline; 2.5159x over previous
import jax
import jax.numpy as jnp
from jax import lax
from jax.experimental import pallas as pl
from jax.experimental.pallas import tpu as pltpu

N_DEV = 4


def kernel(x, dy):
    m, d = x.shape
    _, f = dy.shape
    chunk = d // N_DEV
    f2 = f // 2

    def body(x_ref, dy_ref, out_ref, acc_ref,
             send_r, recv_r, send_l, recv_l,
             ssem_r, rsem_r, ssem_l, rsem_l):
        my = lax.axis_index("i")
        left = lax.rem(my + N_DEV - 1, N_DEV)
        right = lax.rem(my + 1, N_DEV)

        barrier_sem = pltpu.get_barrier_semaphore()
        for nbr in [left, right]:
            pl.semaphore_signal(
                barrier_sem, inc=1,
                device_id=(nbr,), device_id_type=pl.DeviceIdType.MESH,
            )
        pl.semaphore_wait(barrier_sem, 2)

        xb = x_ref[...].astype(jnp.bfloat16)
        dyb = dy_ref[...].astype(jnp.bfloat16)
        acc_ref[...] = lax.dot_general(
            xb, dyb,
            dimension_numbers=(((0,), (0,)), ((), ())),
            preferred_element_type=jnp.float32,
        )

        def start_hop(h):
            scr = lax.rem(my + 2 * N_DEV - 1 - h, N_DEV)
            rdma_r = pltpu.make_async_remote_copy(
                src_ref=send_r.at[h],
                dst_ref=recv_r.at[h],
                send_sem=ssem_r.at[h],
                recv_sem=rsem_r.at[h],
                device_id=(right,),
                device_id_type=pl.DeviceIdType.MESH,
            )
            rdma_l = pltpu.make_async_remote_copy(
                src_ref=send_l.at[h],
                dst_ref=recv_l.at[h],
                send_sem=ssem_l.at[h],
                recv_sem=rsem_l.at[h],
                device_id=(left,),
                device_id_type=pl.DeviceIdType.MESH,
            )
            rdma_r.start()
            rdma_l.start()
            return rdma_r, rdma_l

        scr0 = lax.rem(my + N_DEV - 1, N_DEV)
        scl0 = lax.rem(my + 1, N_DEV)
        send_r[0] = acc_ref[pl.ds(scr0 * chunk, chunk), :f2].astype(jnp.bfloat16)
        send_l[0] = acc_ref[pl.ds(scl0 * chunk, chunk), f2:].astype(jnp.bfloat16)
        rdma_r, rdma_l = start_hop(0)

        for h in range(N_DEV - 1):
            rdma_r.wait()
            rdma_l.wait()
            rcr = lax.rem(my + 2 * N_DEV - 2 - h, N_DEV)
            rcl = lax.rem(my + 2 + h, N_DEV)
            sum_r = (acc_ref[pl.ds(rcr * chunk, chunk), :f2]
                     + recv_r[h].astype(jnp.float32))
            sum_l = (acc_ref[pl.ds(rcl * chunk, chunk), f2:]
                     + recv_l[h].astype(jnp.float32))
            if h < N_DEV - 2:
                send_r[h + 1] = sum_r.astype(jnp.bfloat16)
                send_l[h + 1] = sum_l.astype(jnp.bfloat16)
                rdma_r, rdma_l = start_hop(h + 1)
            else:
                out_ref[:, :f2] = sum_r
                out_ref[:, f2:] = sum_l

    comm = lambda: pltpu.VMEM((N_DEV - 1, chunk, f2), jnp.bfloat16)
    sems = lambda: pltpu.SemaphoreType.DMA((N_DEV - 1,))
    return pl.pallas_call(
        body,
        out_shape=jax.ShapeDtypeStruct((chunk, f), jnp.float32),
        in_specs=[
            pl.BlockSpec(memory_space=pltpu.VMEM),
            pl.BlockSpec(memory_space=pltpu.VMEM),
        ],
        out_specs=pl.BlockSpec(memory_space=pltpu.VMEM),
        scratch_shapes=[
            pltpu.VMEM((d, f), jnp.float32),
            comm(), comm(),
            comm(), comm(),
            sems(), sems(),
            sems(), sems(),
        ],
        compiler_params=pltpu.CompilerParams(
            collective_id=0,
            vmem_limit_bytes=100 * 1024 * 1024,
        ),
    )(x, dy)


# device time: 61357 ns/iter; 2.7316x vs baseline; 1.0857x over previous
import jax
import jax.numpy as jnp
from jax import lax
from jax.experimental import pallas as pl
from jax.experimental.pallas import tpu as pltpu

N_DEV = 4


def kernel(x, dy):
    m, d = x.shape
    _, f = dy.shape
    chunk = d // N_DEV
    f2 = f // 2

    def body(x_ref, dy_ref, out_ref, acc_ref,
             send_r, recv_r, send_l, recv_l,
             ssem_r, rsem_r, ssem_l, rsem_l):
        my = lax.axis_index("i")
        left = lax.rem(my + N_DEV - 1, N_DEV)
        right = lax.rem(my + 1, N_DEV)

        barrier_sem = pltpu.get_barrier_semaphore()
        for nbr in [left, right]:
            pl.semaphore_signal(
                barrier_sem, inc=1,
                device_id=(nbr,), device_id_type=pl.DeviceIdType.MESH,
            )
        pl.semaphore_wait(barrier_sem, 2)

        xb = x_ref[...].astype(jnp.bfloat16)
        dyb = dy_ref[...].astype(jnp.bfloat16)

        def partial(rows, cols_lo):
            xc = x_ref[:, pl.ds(rows * chunk, chunk)].astype(jnp.bfloat16)
            dyc = dy_ref[:, cols_lo:cols_lo + f2].astype(jnp.bfloat16)
            return lax.dot_general(
                xc, dyc,
                dimension_numbers=(((0,), (0,)), ((), ())),
                preferred_element_type=jnp.float32,
            ).astype(jnp.bfloat16)

        def start_hop(h):
            rdma_r = pltpu.make_async_remote_copy(
                src_ref=send_r.at[h], dst_ref=recv_r.at[h],
                send_sem=ssem_r.at[h], recv_sem=rsem_r.at[h],
                device_id=(right,), device_id_type=pl.DeviceIdType.MESH,
            )
            rdma_l = pltpu.make_async_remote_copy(
                src_ref=send_l.at[h], dst_ref=recv_l.at[h],
                send_sem=ssem_l.at[h], recv_sem=rsem_l.at[h],
                device_id=(left,), device_id_type=pl.DeviceIdType.MESH,
            )
            rdma_r.start()
            rdma_l.start()
            return rdma_r, rdma_l

        scr0 = lax.rem(my + N_DEV - 1, N_DEV)
        scl0 = lax.rem(my + 1, N_DEV)
        send_r[0] = partial(scr0, 0)
        send_l[0] = partial(scl0, f2)
        rdma_r, rdma_l = start_hop(0)

        acc_ref[...] = lax.dot_general(
            xb, dyb,
            dimension_numbers=(((0,), (0,)), ((), ())),
            preferred_element_type=jnp.float32,
        ).astype(jnp.bfloat16)

        for h in range(N_DEV - 1):
            rdma_r.wait()
            rdma_l.wait()
            rcr = lax.rem(my + 2 * N_DEV - 2 - h, N_DEV)
            rcl = lax.rem(my + 2 + h, N_DEV)
            sum_r = (acc_ref[pl.ds(rcr * chunk, chunk), :f2].astype(jnp.float32)
                     + recv_r[h].astype(jnp.float32))
            sum_l = (acc_ref[pl.ds(rcl * chunk, chunk), f2:].astype(jnp.float32)
                     + recv_l[h].astype(jnp.float32))
            if h < N_DEV - 2:
                send_r[h + 1] = sum_r.astype(jnp.bfloat16)
                send_l[h + 1] = sum_l.astype(jnp.bfloat16)
                rdma_r, rdma_l = start_hop(h + 1)
            else:
                out_ref[:, :f2] = sum_r
                out_ref[:, f2:] = sum_l

    comm = lambda: pltpu.VMEM((N_DEV - 1, chunk, f2), jnp.bfloat16)
    sems = lambda: pltpu.SemaphoreType.DMA((N_DEV - 1,))
    return pl.pallas_call(
        body,
        out_shape=jax.ShapeDtypeStruct((chunk, f), jnp.float32),
        in_specs=[
            pl.BlockSpec(memory_space=pltpu.VMEM),
            pl.BlockSpec(memory_space=pltpu.VMEM),
        ],
        out_specs=pl.BlockSpec(memory_space=pltpu.VMEM),
        scratch_shapes=[
            pltpu.VMEM((d, f), jnp.bfloat16),
            comm(), comm(),
            comm(), comm(),
            sems(), sems(),
            sems(), sems(),
        ],
        compiler_params=pltpu.CompilerParams(
            collective_id=0,
            vmem_limit_bytes=100 * 1024 * 1024,
        ),
    )(x, dy)


# device time: 53881 ns/iter; 3.1106x vs baseline; 1.1388x over previous
import jax
import jax.numpy as jnp
from jax import lax
from jax.experimental import pallas as pl
from jax.experimental.pallas import tpu as pltpu

N_DEV = 4
SEG = 2


def kernel(x, dy):
    m, d = x.shape
    _, f = dy.shape
    chunk = d // N_DEV
    f2 = f // 2
    fs = f2 // SEG

    def body(x_ref, dy_ref, out_ref, acc_ref,
             send_r, recv_r, send_l, recv_l,
             ssem_r, rsem_r, ssem_l, rsem_l):
        my = lax.axis_index("i")
        left = lax.rem(my + N_DEV - 1, N_DEV)
        right = lax.rem(my + 1, N_DEV)

        barrier_sem = pltpu.get_barrier_semaphore()
        for nbr in [left, right]:
            pl.semaphore_signal(
                barrier_sem, inc=1,
                device_id=(nbr,), device_id_type=pl.DeviceIdType.MESH,
            )
        pl.semaphore_wait(barrier_sem, 2)

        def partial(rows, cols_lo, width):
            xc = x_ref[:, pl.ds(rows * chunk, chunk)].astype(jnp.bfloat16)
            dyc = dy_ref[:, cols_lo:cols_lo + width].astype(jnp.bfloat16)
            return lax.dot_general(
                xc, dyc,
                dimension_numbers=(((0,), (0,)), ((), ())),
                preferred_element_type=jnp.float32,
            ).astype(jnp.bfloat16)

        def start_hop(h, s):
            rdma_r = pltpu.make_async_remote_copy(
                src_ref=send_r.at[h, s], dst_ref=recv_r.at[h, s],
                send_sem=ssem_r.at[h, s], recv_sem=rsem_r.at[h, s],
                device_id=(right,), device_id_type=pl.DeviceIdType.MESH,
            )
            rdma_l = pltpu.make_async_remote_copy(
                src_ref=send_l.at[h, s], dst_ref=recv_l.at[h, s],
                send_sem=ssem_l.at[h, s], recv_sem=rsem_l.at[h, s],
                device_id=(left,), device_id_type=pl.DeviceIdType.MESH,
            )
            rdma_r.start()
            rdma_l.start()
            return rdma_r, rdma_l

        scr0 = lax.rem(my + N_DEV - 1, N_DEV)
        scl0 = lax.rem(my + 1, N_DEV)
        rdmas = []
        for s in range(SEG):
            send_r[0, s] = partial(scr0, s * fs, fs)
            send_l[0, s] = partial(scl0, f2 + s * fs, fs)
            rdmas.append(start_hop(0, s))

        xb = x_ref[...].astype(jnp.bfloat16)
        dyb = dy_ref[...].astype(jnp.bfloat16)
        acc_ref[...] = lax.dot_general(
            xb, dyb,
            dimension_numbers=(((0,), (0,)), ((), ())),
            preferred_element_type=jnp.float32,
        ).astype(jnp.bfloat16)

        for h in range(N_DEV - 1):
            rcr = lax.rem(my + 2 * N_DEV - 2 - h, N_DEV)
            rcl = lax.rem(my + 2 + h, N_DEV)
            next_rdmas = []
            for s in range(SEG):
                rdma_r, rdma_l = rdmas[s]
                rdma_r.wait()
                rdma_l.wait()
                sum_r = (
                    acc_ref[pl.ds(rcr * chunk, chunk),
                            s * fs:(s + 1) * fs].astype(jnp.float32)
                    + recv_r[h, s].astype(jnp.float32)
                )
                sum_l = (
                    acc_ref[pl.ds(rcl * chunk, chunk),
                            f2 + s * fs:f2 + (s + 1) * fs].astype(jnp.float32)
                    + recv_l[h, s].astype(jnp.float32)
                )
                if h < N_DEV - 2:
                    send_r[h + 1, s] = sum_r.astype(jnp.bfloat16)
                    send_l[h + 1, s] = sum_l.astype(jnp.bfloat16)
                    next_rdmas.append(start_hop(h + 1, s))
                else:
                    out_ref[:, s * fs:(s + 1) * fs] = sum_r
                    out_ref[:, f2 + s * fs:f2 + (s + 1) * fs] = sum_l
            rdmas = next_rdmas

    comm = lambda: pltpu.VMEM((N_DEV - 1, SEG, chunk, fs), jnp.bfloat16)
    sems = lambda: pltpu.SemaphoreType.DMA((N_DEV - 1, SEG))
    return pl.pallas_call(
        body,
        out_shape=jax.ShapeDtypeStruct((chunk, f), jnp.float32),
        in_specs=[
            pl.BlockSpec(memory_space=pltpu.VMEM),
            pl.BlockSpec(memory_space=pltpu.VMEM),
        ],
        out_specs=pl.BlockSpec(memory_space=pltpu.VMEM),
        scratch_shapes=[
            pltpu.VMEM((d, f), jnp.bfloat16),
            comm(), comm(),
            comm(), comm(),
            sems(), sems(),
            sems(), sems(),
        ],
        compiler_params=pltpu.CompilerParams(
            collective_id=0,
            vmem_limit_bytes=100 * 1024 * 1024,
        ),
    )(x, dy)
